# Initial kernel scaffold; baseline (speedup 1.0000x reference)
import jax
import jax.numpy as jnp
from jax.experimental import pallas as pl


def kernel(pred_logits, pred_boxes, target_sizes, positive_map):
    raise NotImplementedError("write your pallas kernel here")


# speedup vs baseline: 2.0929x; 2.0929x over previous
import numpy as np
import jax
import jax.numpy as jnp
from jax.experimental import pallas as pl
from jax.experimental.pallas import tpu as pltpu

_TOKEN_IDX = np.array([0, 9, 19, 25, 38, 49, 55, 63, 71, 78, 94, 109, 121,
                       137, 145, 152, 158, 164, 172, 180, 186, 197, 204, 212,
                       222, 233, 244, 254, 260, 271, 281, 288, 300, 314, 321,
                       336, 353, 366, 378, 394, 403, 416, 422, 429, 437, 445,
                       452, 461, 469, 480, 489, 500, 509, 519, 527, 535, 542,
                       550, 558, 573, 579, 594, 603, 608, 617, 625, 634, 645,
                       658, 670, 677, 687, 694, 709, 716, 724, 731, 742, 755,
                       768], dtype=np.int64)
_MAX_TOKEN = 768
_LOOKUP = np.full(_MAX_TOKEN + 1, -1, dtype=np.int64)
_LOOKUP[_TOKEN_IDX] = np.arange(len(_TOKEN_IDX), dtype=np.int64)
_CLS_TABLE = np.maximum(_LOOKUP, 0).astype(np.int32)

_B = 64
_Q = 900
_C = 256
_T = _MAX_TOKEN + 1
_KR = 64
_K = 50
_DCH = 8


def _rowmax_kernel(logits_ref, pm_ref, rmax_ref):
    x = jax.nn.sigmoid(logits_ref[0])
    prob = jax.lax.dot_general(x, pm_ref[...], (((1,), (1,)), ((), ())),
                               preferred_element_type=jnp.float32)
    rmax_ref[0] = jnp.max(prob, axis=1, keepdims=True)


def _row_topk_kernel(rmax_ref, rows_ref):
    m0 = rmax_ref[...]
    iota = jax.lax.broadcasted_iota(jnp.int32, (_B, _Q), 1)
    lane = jax.lax.broadcasted_iota(jnp.int32, (_B, _KR), 1)

    def body(i, carry):
        m, rows = carry
        mx = jnp.max(m, axis=1, keepdims=True)
        idx = jnp.min(jnp.where(m == mx, iota, _Q), axis=1, keepdims=True)
        rows = jnp.where(lane == i, idx, rows)
        m = jnp.where(iota == idx, -jnp.inf, m)
        return m, rows

    _, rows = jax.lax.fori_loop(
        0, _KR, body, (m0, jnp.zeros((_B, _KR), jnp.int32)))
    rows_ref[...] = rows


def _cand_prob_kernel(rows_sref, logits_ref, pm_ref, out_ref, prob_s):
    b = pl.program_id(0)
    x = jax.nn.sigmoid(logits_ref[0])
    prob_s[...] = jax.lax.dot_general(x, pm_ref[...], (((1,), (1,)), ((), ())),
                                      preferred_element_type=jnp.float32)

    def copy_body(i, _):
        r = rows_sref[b * _KR + i]
        out_ref[0, pl.ds(i, 1), :] = prob_s[pl.ds(r, 1), :]
        return 0

    jax.lax.fori_loop(0, _KR, copy_body, 0)


def _final_kernel(cand_ref, rows_ref, boxes_ref, ts_ref, cls_ref,
                  scores_ref, labels_ref, boxes_out_ref):
    n = _DCH
    p0 = cand_ref[...]
    rows = rows_ref[...]
    lanes = jax.lax.broadcasted_iota(jnp.int32, (n, _KR, _T), 2)
    flat = rows[:, :, None] * _T + lanes
    lane50 = jax.lax.broadcasted_iota(jnp.int32, (n, _K), 1)
    big = _Q * _T

    def topk_body(i, carry):
        p, sc, ix = carry
        mx = jnp.max(jnp.max(p, axis=2, keepdims=True), axis=1, keepdims=True)
        sel = jnp.min(jnp.min(jnp.where(p == mx, flat, big),
                              axis=2, keepdims=True), axis=1, keepdims=True)
        sc = jnp.where(lane50 == i, mx[:, :, 0], sc)
        ix = jnp.where(lane50 == i, sel[:, :, 0], ix)
        p = jnp.where(flat == sel, -jnp.inf, p)
        return p, sc, ix

    _, sc, ix = jax.lax.fori_loop(
        0, _K, topk_body,
        (p0, jnp.zeros((n, _K), jnp.float32), jnp.zeros((n, _K), jnp.int32)))

    j = ix // _T
    lab = ix - j * _T

    table = cls_ref[...]
    pb = boxes_ref[...]
    cx, cy, w, h = pb[:, :, 0], pb[:, :, 1], pb[:, :, 2], pb[:, :, 3]
    x0 = cx - 0.5 * w
    y0 = cy - 0.5 * h
    x1 = cx + 0.5 * w
    y1 = cy + 0.5 * h
    ts = ts_ref[...].astype(jnp.float32)
    ih = ts[:, 0:1]
    iw = ts[:, 1:2]

    q_iota = jax.lax.broadcasted_iota(jnp.int32, (n, _Q), 1)
    t_iota = jax.lax.broadcasted_iota(jnp.int32, (n, _T), 1)

    def gather_body(i, carry):
        cls_a, b0, b1, b2, b3 = carry
        rowi = jnp.sum(jnp.where(lane50 == i, j, 0), axis=1, keepdims=True)
        labi = jnp.sum(jnp.where(lane50 == i, lab, 0), axis=1, keepdims=True)
        clsi = jnp.sum(jnp.where(t_iota == labi, table, 0), axis=1, keepdims=True)
        mq = q_iota == rowi
        g0 = jnp.sum(jnp.where(mq, x0, 0.0), axis=1, keepdims=True)
        g1 = jnp.sum(jnp.where(mq, y0, 0.0), axis=1, keepdims=True)
        g2 = jnp.sum(jnp.where(mq, x1, 0.0), axis=1, keepdims=True)
        g3 = jnp.sum(jnp.where(mq, y1, 0.0), axis=1, keepdims=True)
        cls_a = jnp.where(lane50 == i, clsi, cls_a)
        b0 = jnp.where(lane50 == i, g0, b0)
        b1 = jnp.where(lane50 == i, g1, b1)
        b2 = jnp.where(lane50 == i, g2, b2)
        b3 = jnp.where(lane50 == i, g3, b3)
        return cls_a, b0, b1, b2, b3

    zf = jnp.zeros((n, _K), jnp.float32)
    cls_a, b0, b1, b2, b3 = jax.lax.fori_loop(
        0, _K, gather_body,
        (jnp.zeros((n, _K), jnp.int32), zf, zf, zf, zf))

    scores_ref[...] = sc
    labels_ref[...] = cls_a
    boxes_out_ref[...] = jnp.stack(
        [b0 * iw, b1 * ih, b2 * iw, b3 * ih], axis=-1)


def kernel(pred_logits, pred_boxes, target_sizes, positive_map):
    rmax = pl.pallas_call(
        _rowmax_kernel,
        grid=(_B,),
        in_specs=[
            pl.BlockSpec((1, _Q, _C), lambda b: (b, 0, 0)),
            pl.BlockSpec((_T, _C), lambda b: (0, 0)),
        ],
        out_specs=pl.BlockSpec((1, _Q, 1), lambda b: (b, 0, 0)),
        out_shape=jax.ShapeDtypeStruct((_B, _Q, 1), jnp.float32),
    )(pred_logits, positive_map)

    rows = pl.pallas_call(
        _row_topk_kernel,
        in_specs=[pl.BlockSpec((_B, _Q), lambda: (0, 0))],
        out_specs=pl.BlockSpec((_B, _KR), lambda: (0, 0)),
        out_shape=jax.ShapeDtypeStruct((_B, _KR), jnp.int32),
    )(rmax.reshape(_B, _Q))

    cand = pl.pallas_call(
        _cand_prob_kernel,
        grid_spec=pltpu.PrefetchScalarGridSpec(
            num_scalar_prefetch=1,
            grid=(_B,),
            in_specs=[
                pl.BlockSpec((1, _Q, _C), lambda b, sref: (b, 0, 0)),
                pl.BlockSpec((_T, _C), lambda b, sref: (0, 0)),
            ],
            out_specs=pl.BlockSpec((1, _KR, _T), lambda b, sref: (b, 0, 0)),
            scratch_shapes=[pltpu.VMEM((_Q, _T), jnp.float32)],
        ),
        out_shape=jax.ShapeDtypeStruct((_B, _KR, _T), jnp.float32),
    )(rows.reshape(_B * _KR), pred_logits, positive_map)

    cls_table = jnp.asarray(_CLS_TABLE).reshape(1, _T)
    scores, labels, boxes = pl.pallas_call(
        _final_kernel,
        grid=(_B // _DCH,),
        in_specs=[
            pl.BlockSpec((_DCH, _KR, _T), lambda b: (b, 0, 0)),
            pl.BlockSpec((_DCH, _KR), lambda b: (b, 0)),
            pl.BlockSpec((_DCH, _Q, 4), lambda b: (b, 0, 0)),
            pl.BlockSpec((_DCH, 2), lambda b: (b, 0)),
            pl.BlockSpec((1, _T), lambda b: (0, 0)),
        ],
        out_specs=[
            pl.BlockSpec((_DCH, _K), lambda b: (b, 0)),
            pl.BlockSpec((_DCH, _K), lambda b: (b, 0)),
            pl.BlockSpec((_DCH, _K, 4), lambda b: (b, 0, 0)),
        ],
        out_shape=[
            jax.ShapeDtypeStruct((_B, _K), jnp.float32),
            jax.ShapeDtypeStruct((_B, _K), jnp.int32),
            jax.ShapeDtypeStruct((_B, _K, 4), jnp.float32),
        ],
    )(cand, rows, pred_boxes, target_sizes, cls_table)

    return scores, labels, boxes


# speedup vs baseline: 7.2064x; 3.4432x over previous
import numpy as np
import jax
import jax.numpy as jnp
from jax.experimental import pallas as pl
from jax.experimental.pallas import tpu as pltpu

_TOKEN_IDX = np.array([0, 9, 19, 25, 38, 49, 55, 63, 71, 78, 94, 109, 121,
                       137, 145, 152, 158, 164, 172, 180, 186, 197, 204, 212,
                       222, 233, 244, 254, 260, 271, 281, 288, 300, 314, 321,
                       336, 353, 366, 378, 394, 403, 416, 422, 429, 437, 445,
                       452, 461, 469, 480, 489, 500, 509, 519, 527, 535, 542,
                       550, 558, 573, 579, 594, 603, 608, 617, 625, 634, 645,
                       658, 670, 677, 687, 694, 709, 716, 724, 731, 742, 755,
                       768], dtype=np.int64)
_MAX_TOKEN = 768
_LOOKUP = np.full(_MAX_TOKEN + 1, -1, dtype=np.int64)
_LOOKUP[_TOKEN_IDX] = np.arange(len(_TOKEN_IDX), dtype=np.int64)
_CLS_TABLE = np.maximum(_LOOKUP, 0).astype(np.int32)

_B = 64
_Q = 900
_C = 256
_T = _MAX_TOKEN + 1
_KR = 64
_K = 50
_DCH = 8


def _rowmax_kernel(logits_ref, pm_ref, rmax_ref):
    x = jax.nn.sigmoid(logits_ref[0])
    prob = jax.lax.dot_general(x, pm_ref[...], (((1,), (1,)), ((), ())),
                               preferred_element_type=jnp.float32)
    rmax_ref[0] = jnp.max(prob, axis=1, keepdims=True)


def _row_topk_kernel(rmax_ref, rows_ref):
    m0 = rmax_ref[...]
    iota = jax.lax.broadcasted_iota(jnp.int32, (_B, _Q), 1)
    lane = jax.lax.broadcasted_iota(jnp.int32, (_B, _KR), 1)

    def body(i, carry):
        m, rows = carry
        mx = jnp.max(m, axis=1, keepdims=True)
        idx = jnp.min(jnp.where(m == mx, iota, _Q), axis=1, keepdims=True)
        rows = jnp.where(lane == i, idx, rows)
        m = jnp.where(iota == idx, -jnp.inf, m)
        return m, rows

    _, rows = jax.lax.fori_loop(
        0, _KR, body, (m0, jnp.zeros((_B, _KR), jnp.int32)))
    rows_ref[...] = rows


def _cand_prob_kernel(rows_sref, logits_ref, pm_ref, out_ref, prob_s):
    b = pl.program_id(0)
    x = jax.nn.sigmoid(logits_ref[0])
    prob_s[...] = jax.lax.dot_general(x, pm_ref[...], (((1,), (1,)), ((), ())),
                                      preferred_element_type=jnp.float32)

    def copy_body(i, _):
        r = rows_sref[b * _KR + i]
        out_ref[0, pl.ds(i, 1), :] = prob_s[pl.ds(r, 1), :]
        return 0

    jax.lax.fori_loop(0, _KR, copy_body, 0)


_NW = 25
_ICH = 8


def _rowstat_kernel(cand_ref, rows_ref, m_ref, a_ref):
    p = cand_ref[...]
    rows = rows_ref[...]
    lane_t3 = jax.lax.broadcasted_iota(jnp.int32, (_ICH, _KR, _T), 2)
    m0 = jnp.max(p, axis=2)
    col = jnp.min(jnp.where(p == m0[:, :, None], lane_t3, _T), axis=2)
    m_ref[...] = m0
    a_ref[...] = rows * _T + col


def _final_kernel(candf_ref, m_ref, a_ref, rowsf_ref, boxes_ref, ts_ref,
                  cls_ref, scores_ref, labels_ref, boxes_out_ref):
    n = _B
    m0 = m_ref[...]
    a0 = a_ref[...]
    rowsf = rowsf_ref[...]
    lane50 = jax.lax.broadcasted_iota(jnp.int32, (n, _K), 1)
    lane_t = jax.lax.broadcasted_iota(jnp.int32, (n, _T), 1)
    w_iota = jax.lax.broadcasted_iota(jnp.int32, (n, _KR, _NW), 2)
    big = _Q * _T

    g_iota = jax.lax.broadcasted_iota(jnp.int32, (n, n * _KR), 1)
    b_iota = jax.lax.broadcasted_iota(jnp.int32, (n, n * _KR), 0)
    eqb = (g_iota // _KR) == b_iota

    def topk_body(i, carry):
        m, a, taken, sc, ix = carry
        pf = candf_ref[...]
        mx = jnp.max(m, axis=1, keepdims=True)
        sel = jnp.min(jnp.where(m == mx, a, big), axis=1, keepdims=True)
        sc = jnp.where(lane50 == i, mx, sc)
        ix = jnp.where(lane50 == i, sel, ix)
        selrow = sel // _T
        selcol = sel - selrow * _T
        jhot = (m == mx) & (a == sel)
        jhotf = (eqb & (rowsf == selrow)).astype(jnp.float32)
        xrow = jax.lax.dot_general(jhotf, pf, (((1,), (0,)), ((), ())),
                                   precision=jax.lax.Precision.HIGHEST,
                                   preferred_element_type=jnp.float32)
        jhot3 = (m[:, :, None] == mx[:, :, None]) & (a[:, :, None] == sel[:, :, None])
        tw = jnp.sum(jnp.where(jhot3, taken, 0), axis=1)
        exp = jnp.broadcast_to(tw[:, :, None], (n, _NW, 32))
        exp = exp.reshape(n, _NW * 32)[:, :_T]
        bit = jax.lax.shift_right_logical(exp, lane_t % 32) & 1
        dead = (bit == 1) | (lane_t == selcol)
        xm = jnp.where(dead, -jnp.inf, xrow)
        newmax = jnp.max(xm, axis=1, keepdims=True)
        flatx = selrow * _T + lane_t
        newa = jnp.min(jnp.where(xm == newmax, flatx, big), axis=1,
                       keepdims=True)
        m = jnp.where(jhot, newmax, m)
        a = jnp.where(jhot, newa, a)
        setmask = jhot3 & (w_iota == (selcol // 32)[:, :, None])
        bitval = jax.lax.shift_left(jnp.int32(1), (selcol % 32)[:, :, None])
        taken = taken | jnp.where(setmask, bitval, 0)
        return m, a, taken, sc, ix

    _, _, _, sc, ix = jax.lax.fori_loop(
        0, _K, topk_body,
        (m0, a0, jnp.zeros((n, _KR, _NW), jnp.int32),
         jnp.zeros((n, _K), jnp.float32), jnp.zeros((n, _K), jnp.int32)))

    j = ix // _T
    lab = ix - j * _T

    table = cls_ref[...]
    pb = boxes_ref[...]
    cx, cy, w, h = pb[:, 0, :], pb[:, 1, :], pb[:, 2, :], pb[:, 3, :]
    x0 = cx - 0.5 * w
    y0 = cy - 0.5 * h
    x1 = cx + 0.5 * w
    y1 = cy + 0.5 * h
    ts = ts_ref[...].astype(jnp.float32)
    ih = ts[:, 0:1]
    iw = ts[:, 1:2]

    q_iota = jax.lax.broadcasted_iota(jnp.int32, (n, _Q), 1)
    t_iota = jax.lax.broadcasted_iota(jnp.int32, (n, _T), 1)

    def gather_body(i, carry):
        cls_a, b0, b1, b2, b3 = carry
        rowi = jnp.sum(jnp.where(lane50 == i, j, 0), axis=1, keepdims=True)
        labi = jnp.sum(jnp.where(lane50 == i, lab, 0), axis=1, keepdims=True)
        clsi = jnp.sum(jnp.where(t_iota == labi, table, 0), axis=1, keepdims=True)
        mq = q_iota == rowi
        g0 = jnp.sum(jnp.where(mq, x0, 0.0), axis=1, keepdims=True)
        g1 = jnp.sum(jnp.where(mq, y0, 0.0), axis=1, keepdims=True)
        g2 = jnp.sum(jnp.where(mq, x1, 0.0), axis=1, keepdims=True)
        g3 = jnp.sum(jnp.where(mq, y1, 0.0), axis=1, keepdims=True)
        cls_a = jnp.where(lane50 == i, clsi, cls_a)
        b0 = jnp.where(lane50 == i, g0, b0)
        b1 = jnp.where(lane50 == i, g1, b1)
        b2 = jnp.where(lane50 == i, g2, b2)
        b3 = jnp.where(lane50 == i, g3, b3)
        return cls_a, b0, b1, b2, b3

    zf = jnp.zeros((n, _K), jnp.float32)
    cls_a, b0, b1, b2, b3 = jax.lax.fori_loop(
        0, _K, gather_body,
        (jnp.zeros((n, _K), jnp.int32), zf, zf, zf, zf))

    scores_ref[...] = sc
    labels_ref[...] = cls_a
    boxes_out_ref[...] = jnp.stack(
        [b0 * iw, b1 * ih, b2 * iw, b3 * ih], axis=-1)


def kernel(pred_logits, pred_boxes, target_sizes, positive_map):
    rmax = pl.pallas_call(
        _rowmax_kernel,
        grid=(_B,),
        in_specs=[
            pl.BlockSpec((1, _Q, _C), lambda b: (b, 0, 0)),
            pl.BlockSpec((_T, _C), lambda b: (0, 0)),
        ],
        out_specs=pl.BlockSpec((1, _Q, 1), lambda b: (b, 0, 0)),
        out_shape=jax.ShapeDtypeStruct((_B, _Q, 1), jnp.float32),
    )(pred_logits, positive_map)

    rows = pl.pallas_call(
        _row_topk_kernel,
        in_specs=[pl.BlockSpec((_B, _Q), lambda: (0, 0))],
        out_specs=pl.BlockSpec((_B, _KR), lambda: (0, 0)),
        out_shape=jax.ShapeDtypeStruct((_B, _KR), jnp.int32),
    )(rmax.reshape(_B, _Q))

    cand = pl.pallas_call(
        _cand_prob_kernel,
        grid_spec=pltpu.PrefetchScalarGridSpec(
            num_scalar_prefetch=1,
            grid=(_B,),
            in_specs=[
                pl.BlockSpec((1, _Q, _C), lambda b, sref: (b, 0, 0)),
                pl.BlockSpec((_T, _C), lambda b, sref: (0, 0)),
            ],
            out_specs=pl.BlockSpec((1, _KR, _T), lambda b, sref: (b, 0, 0)),
            scratch_shapes=[pltpu.VMEM((_Q, _T), jnp.float32)],
        ),
        out_shape=jax.ShapeDtypeStruct((_B, _KR, _T), jnp.float32),
    )(rows.reshape(_B * _KR), pred_logits, positive_map)

    m0, a0 = pl.pallas_call(
        _rowstat_kernel,
        grid=(_B // _ICH,),
        in_specs=[
            pl.BlockSpec((_ICH, _KR, _T), lambda b: (b, 0, 0)),
            pl.BlockSpec((_ICH, _KR), lambda b: (b, 0)),
        ],
        out_specs=[
            pl.BlockSpec((_ICH, _KR), lambda b: (b, 0)),
            pl.BlockSpec((_ICH, _KR), lambda b: (b, 0)),
        ],
        out_shape=[
            jax.ShapeDtypeStruct((_B, _KR), jnp.float32),
            jax.ShapeDtypeStruct((_B, _KR), jnp.int32),
        ],
    )(cand, rows)

    cls_table = jnp.asarray(_CLS_TABLE).reshape(1, _T)
    boxes_t = jnp.transpose(pred_boxes, (0, 2, 1))
    scores, labels, boxes = pl.pallas_call(
        _final_kernel,
        in_specs=[
            pl.BlockSpec((_B * _KR, _T), lambda: (0, 0)),
            pl.BlockSpec((_B, _KR), lambda: (0, 0)),
            pl.BlockSpec((_B, _KR), lambda: (0, 0)),
            pl.BlockSpec((1, _B * _KR), lambda: (0, 0)),
            pl.BlockSpec((_B, 4, _Q), lambda: (0, 0, 0)),
            pl.BlockSpec((_B, 2), lambda: (0, 0)),
            pl.BlockSpec((1, _T), lambda: (0, 0)),
        ],
        out_specs=[
            pl.BlockSpec((_B, _K), lambda: (0, 0)),
            pl.BlockSpec((_B, _K), lambda: (0, 0)),
            pl.BlockSpec((_B, _K, 4), lambda: (0, 0, 0)),
        ],
        out_shape=[
            jax.ShapeDtypeStruct((_B, _K), jnp.float32),
            jax.ShapeDtypeStruct((_B, _K), jnp.int32),
            jax.ShapeDtypeStruct((_B, _K, 4), jnp.float32),
        ],
    )(cand.reshape(_B * _KR, _T), m0, a0, rows.reshape(1, _B * _KR),
      boxes_t, target_sizes, cls_table)

    return scores, labels, boxes


# speedup vs baseline: 7.5556x; 1.0485x over previous
import numpy as np
import jax
import jax.numpy as jnp
from jax.experimental import pallas as pl
from jax.experimental.pallas import tpu as pltpu

_TOKEN_IDX = np.array([0, 9, 19, 25, 38, 49, 55, 63, 71, 78, 94, 109, 121,
                       137, 145, 152, 158, 164, 172, 180, 186, 197, 204, 212,
                       222, 233, 244, 254, 260, 271, 281, 288, 300, 314, 321,
                       336, 353, 366, 378, 394, 403, 416, 422, 429, 437, 445,
                       452, 461, 469, 480, 489, 500, 509, 519, 527, 535, 542,
                       550, 558, 573, 579, 594, 603, 608, 617, 625, 634, 645,
                       658, 670, 677, 687, 694, 709, 716, 724, 731, 742, 755,
                       768], dtype=np.int64)
_MAX_TOKEN = 768
_LOOKUP = np.full(_MAX_TOKEN + 1, -1, dtype=np.int64)
_LOOKUP[_TOKEN_IDX] = np.arange(len(_TOKEN_IDX), dtype=np.int64)
_CLS_TABLE = np.maximum(_LOOKUP, 0).astype(np.int32)

_B = 64
_Q = 900
_C = 256
_T = _MAX_TOKEN + 1
_KR = 64
_K = 50
_DCH = 8


def _rowmax_kernel(logits_ref, pm_ref, rmax_ref):
    x = jax.nn.sigmoid(logits_ref[0])
    prob = jax.lax.dot_general(x, pm_ref[...], (((1,), (1,)), ((), ())),
                               preferred_element_type=jnp.float32)
    rmax_ref[0] = jnp.max(prob, axis=1, keepdims=True)


def _row_topk_kernel(rmax_ref, rows_ref):
    m0 = rmax_ref[...]
    iota = jax.lax.broadcasted_iota(jnp.int32, (_B, _Q), 1)
    lane = jax.lax.broadcasted_iota(jnp.int32, (_B, _KR), 1)

    def body(i, carry):
        m, rows = carry
        mx = jnp.max(m, axis=1, keepdims=True)
        idx = jnp.min(jnp.where(m == mx, iota, _Q), axis=1, keepdims=True)
        rows = jnp.where(lane == i, idx, rows)
        m = jnp.where(iota == idx, -jnp.inf, m)
        return m, rows

    _, rows = jax.lax.fori_loop(
        0, _KR, body, (m0, jnp.zeros((_B, _KR), jnp.int32)))
    rows_ref[...] = rows


def _cand_prob_kernel(rows_sref, logits_ref, pm_ref, out_ref, prob_s):
    b = pl.program_id(0)
    x = jax.nn.sigmoid(logits_ref[0])
    prob_s[...] = jax.lax.dot_general(x, pm_ref[...], (((1,), (1,)), ((), ())),
                                      preferred_element_type=jnp.float32)

    def copy_body(i, _):
        r = rows_sref[b * _KR + i]
        out_ref[0, pl.ds(i, 1), :] = prob_s[pl.ds(r, 1), :]
        return 0

    jax.lax.fori_loop(0, _KR, copy_body, 0)


_NW = 25
_ICH = 8


def _rowstat_kernel(cand_ref, rows_ref, m_ref, a_ref):
    p = cand_ref[...]
    rows = rows_ref[...]
    lane_t3 = jax.lax.broadcasted_iota(jnp.int32, (_ICH, _KR, _T), 2)
    m0 = jnp.max(p, axis=2)
    col = jnp.min(jnp.where(p == m0[:, :, None], lane_t3, _T), axis=2)
    m_ref[...] = m0
    a_ref[...] = rows * _T + col


def _final_kernel(candf_ref, m_ref, a_ref, rowsf_ref,
                  scores_ref, ix_ref):
    n = _B
    m0 = m_ref[...]
    a0 = a_ref[...]
    rowsf = rowsf_ref[...]
    lane50 = jax.lax.broadcasted_iota(jnp.int32, (n, _K), 1)
    lane_t = jax.lax.broadcasted_iota(jnp.int32, (n, _T), 1)
    w_iota = jax.lax.broadcasted_iota(jnp.int32, (n, _KR, _NW), 2)
    big = _Q * _T

    g_iota = jax.lax.broadcasted_iota(jnp.int32, (n, n * _KR), 1)
    b_iota = jax.lax.broadcasted_iota(jnp.int32, (n, n * _KR), 0)
    eqb = (g_iota // _KR) == b_iota

    def topk_body(i, carry):
        m, a, taken, sc, ix = carry
        pf = candf_ref[...]
        mx = jnp.max(m, axis=1, keepdims=True)
        sel = jnp.min(jnp.where(m == mx, a, big), axis=1, keepdims=True)
        sc = jnp.where(lane50 == i, mx, sc)
        ix = jnp.where(lane50 == i, sel, ix)
        selrow = sel // _T
        selcol = sel - selrow * _T
        jhot = (m == mx) & (a == sel)
        jhotf = (eqb & (rowsf == selrow)).astype(jnp.float32)
        xrow = jax.lax.dot_general(jhotf, pf, (((1,), (0,)), ((), ())),
                                   precision=jax.lax.Precision.HIGHEST,
                                   preferred_element_type=jnp.float32)
        jhot3 = (m[:, :, None] == mx[:, :, None]) & (a[:, :, None] == sel[:, :, None])
        tw = jnp.sum(jnp.where(jhot3, taken, 0), axis=1)
        exp = jnp.broadcast_to(tw[:, :, None], (n, _NW, 32))
        exp = exp.reshape(n, _NW * 32)[:, :_T]
        bit = jax.lax.shift_right_logical(exp, lane_t % 32) & 1
        dead = (bit == 1) | (lane_t == selcol)
        xm = jnp.where(dead, -jnp.inf, xrow)
        newmax = jnp.max(xm, axis=1, keepdims=True)
        flatx = selrow * _T + lane_t
        newa = jnp.min(jnp.where(xm == newmax, flatx, big), axis=1,
                       keepdims=True)
        m = jnp.where(jhot, newmax, m)
        a = jnp.where(jhot, newa, a)
        setmask = jhot3 & (w_iota == (selcol // 32)[:, :, None])
        bitval = jax.lax.shift_left(jnp.int32(1), (selcol % 32)[:, :, None])
        taken = taken | jnp.where(setmask, bitval, 0)
        return m, a, taken, sc, ix

    _, _, _, sc, ix = jax.lax.fori_loop(
        0, _K, topk_body,
        (m0, a0, jnp.zeros((n, _KR, _NW), jnp.int32),
         jnp.zeros((n, _K), jnp.float32), jnp.zeros((n, _K), jnp.int32)))

    scores_ref[...] = sc
    ix_ref[...] = ix


def _gather_kernel(ix_ref, boxes_ref, ts_ref, cls_ref,
                   labels_ref, boxes_out_ref):
    n = _ICH
    ix = ix_ref[...]
    j = ix // _T
    lab = ix - j * _T

    table = cls_ref[...]
    pb = boxes_ref[...]
    cx, cy, w, h = pb[:, 0, :], pb[:, 1, :], pb[:, 2, :], pb[:, 3, :]
    x0 = cx - 0.5 * w
    y0 = cy - 0.5 * h
    x1 = cx + 0.5 * w
    y1 = cy + 0.5 * h
    ts = ts_ref[...].astype(jnp.float32)
    ih = ts[:, 0:1]
    iw = ts[:, 1:2]

    q_iota = jax.lax.broadcasted_iota(jnp.int32, (n, _K, _Q), 2)
    t_iota = jax.lax.broadcasted_iota(jnp.int32, (n, _K, _T), 2)
    mq = q_iota == j[:, :, None]
    mt = t_iota == lab[:, :, None]
    cls_a = jnp.sum(jnp.where(mt, table[:, None, :], 0), axis=2)
    b0 = jnp.sum(jnp.where(mq, x0[:, None, :], 0.0), axis=2) * iw
    b1 = jnp.sum(jnp.where(mq, y0[:, None, :], 0.0), axis=2) * ih
    b2 = jnp.sum(jnp.where(mq, x1[:, None, :], 0.0), axis=2) * iw
    b3 = jnp.sum(jnp.where(mq, y1[:, None, :], 0.0), axis=2) * ih

    labels_ref[...] = cls_a
    boxes_out_ref[...] = jnp.stack([b0, b1, b2, b3], axis=-1)


def kernel(pred_logits, pred_boxes, target_sizes, positive_map):
    rmax = pl.pallas_call(
        _rowmax_kernel,
        grid=(_B,),
        in_specs=[
            pl.BlockSpec((1, _Q, _C), lambda b: (b, 0, 0)),
            pl.BlockSpec((_T, _C), lambda b: (0, 0)),
        ],
        out_specs=pl.BlockSpec((1, _Q, 1), lambda b: (b, 0, 0)),
        out_shape=jax.ShapeDtypeStruct((_B, _Q, 1), jnp.float32),
    )(pred_logits, positive_map)

    rows = pl.pallas_call(
        _row_topk_kernel,
        in_specs=[pl.BlockSpec((_B, _Q), lambda: (0, 0))],
        out_specs=pl.BlockSpec((_B, _KR), lambda: (0, 0)),
        out_shape=jax.ShapeDtypeStruct((_B, _KR), jnp.int32),
    )(rmax.reshape(_B, _Q))

    cand = pl.pallas_call(
        _cand_prob_kernel,
        grid_spec=pltpu.PrefetchScalarGridSpec(
            num_scalar_prefetch=1,
            grid=(_B,),
            in_specs=[
                pl.BlockSpec((1, _Q, _C), lambda b, sref: (b, 0, 0)),
                pl.BlockSpec((_T, _C), lambda b, sref: (0, 0)),
            ],
            out_specs=pl.BlockSpec((1, _KR, _T), lambda b, sref: (b, 0, 0)),
            scratch_shapes=[pltpu.VMEM((_Q, _T), jnp.float32)],
        ),
        out_shape=jax.ShapeDtypeStruct((_B, _KR, _T), jnp.float32),
    )(rows.reshape(_B * _KR), pred_logits, positive_map)

    m0, a0 = pl.pallas_call(
        _rowstat_kernel,
        grid=(_B // _ICH,),
        in_specs=[
            pl.BlockSpec((_ICH, _KR, _T), lambda b: (b, 0, 0)),
            pl.BlockSpec((_ICH, _KR), lambda b: (b, 0)),
        ],
        out_specs=[
            pl.BlockSpec((_ICH, _KR), lambda b: (b, 0)),
            pl.BlockSpec((_ICH, _KR), lambda b: (b, 0)),
        ],
        out_shape=[
            jax.ShapeDtypeStruct((_B, _KR), jnp.float32),
            jax.ShapeDtypeStruct((_B, _KR), jnp.int32),
        ],
    )(cand, rows)

    scores, ix = pl.pallas_call(
        _final_kernel,
        in_specs=[
            pl.BlockSpec((_B * _KR, _T), lambda: (0, 0)),
            pl.BlockSpec((_B, _KR), lambda: (0, 0)),
            pl.BlockSpec((_B, _KR), lambda: (0, 0)),
            pl.BlockSpec((1, _B * _KR), lambda: (0, 0)),
        ],
        out_specs=[
            pl.BlockSpec((_B, _K), lambda: (0, 0)),
            pl.BlockSpec((_B, _K), lambda: (0, 0)),
        ],
        out_shape=[
            jax.ShapeDtypeStruct((_B, _K), jnp.float32),
            jax.ShapeDtypeStruct((_B, _K), jnp.int32),
        ],
    )(cand.reshape(_B * _KR, _T), m0, a0, rows.reshape(1, _B * _KR))

    cls_table = jnp.asarray(_CLS_TABLE).reshape(1, _T)
    boxes_t = jnp.transpose(pred_boxes, (0, 2, 1))
    labels, boxes = pl.pallas_call(
        _gather_kernel,
        grid=(_B // _ICH,),
        in_specs=[
            pl.BlockSpec((_ICH, _K), lambda b: (b, 0)),
            pl.BlockSpec((_ICH, 4, _Q), lambda b: (b, 0, 0)),
            pl.BlockSpec((_ICH, 2), lambda b: (b, 0)),
            pl.BlockSpec((1, _T), lambda b: (0, 0)),
        ],
        out_specs=[
            pl.BlockSpec((_ICH, _K), lambda b: (b, 0)),
            pl.BlockSpec((_ICH, _K, 4), lambda b: (b, 0, 0)),
        ],
        out_shape=[
            jax.ShapeDtypeStruct((_B, _K), jnp.int32),
            jax.ShapeDtypeStruct((_B, _K, 4), jnp.float32),
        ],
    )(ix, boxes_t, target_sizes, cls_table)

    return scores, labels, boxes


# speedup vs baseline: 10.0393x; 1.3287x over previous
import numpy as np
import jax
import jax.numpy as jnp
from jax.experimental import pallas as pl
from jax.experimental.pallas import tpu as pltpu

_TOKEN_IDX = np.array([0, 9, 19, 25, 38, 49, 55, 63, 71, 78, 94, 109, 121,
                       137, 145, 152, 158, 164, 172, 180, 186, 197, 204, 212,
                       222, 233, 244, 254, 260, 271, 281, 288, 300, 314, 321,
                       336, 353, 366, 378, 394, 403, 416, 422, 429, 437, 445,
                       452, 461, 469, 480, 489, 500, 509, 519, 527, 535, 542,
                       550, 558, 573, 579, 594, 603, 608, 617, 625, 634, 645,
                       658, 670, 677, 687, 694, 709, 716, 724, 731, 742, 755,
                       768], dtype=np.int64)
_MAX_TOKEN = 768
_LOOKUP = np.full(_MAX_TOKEN + 1, -1, dtype=np.int64)
_LOOKUP[_TOKEN_IDX] = np.arange(len(_TOKEN_IDX), dtype=np.int64)
_CLS_TABLE = np.maximum(_LOOKUP, 0).astype(np.int32)

_B = 64
_Q = 900
_C = 256
_T = _MAX_TOKEN + 1
_KR = 64
_K = 50
_DCH = 8


def _rowmax_kernel(logits_ref, pm_ref, rmax_ref):
    x = jax.nn.sigmoid(logits_ref[0])
    prob = jax.lax.dot_general(x, pm_ref[...], (((1,), (1,)), ((), ())),
                               preferred_element_type=jnp.float32)
    rmax_ref[0] = jnp.max(prob, axis=1, keepdims=True)


def _row_topk_kernel(rmax_ref, rows_ref):
    m0 = rmax_ref[...]
    iota = jax.lax.broadcasted_iota(jnp.int32, (_B, _Q), 1)
    lane = jax.lax.broadcasted_iota(jnp.int32, (_B, _KR), 1)

    def body(i, carry):
        m, rows = carry
        mx = jnp.max(m, axis=1, keepdims=True)
        idx = jnp.min(jnp.where(m == mx, iota, _Q), axis=1, keepdims=True)
        rows = jnp.where(lane == i, idx, rows)
        m = jnp.where(iota == idx, -jnp.inf, m)
        return m, rows

    _, rows = jax.lax.fori_loop(
        0, _KR, body, (m0, jnp.zeros((_B, _KR), jnp.int32)))
    rows_ref[...] = rows


def _cand_prob_kernel(rows_sref, logits_ref, pm_ref, out_ref, prob_s):
    b = pl.program_id(0)
    x = jax.nn.sigmoid(logits_ref[0])
    prob_s[...] = jax.lax.dot_general(x, pm_ref[...], (((1,), (1,)), ((), ())),
                                      preferred_element_type=jnp.float32)

    def copy_body(i, _):
        r = rows_sref[b * _KR + i]
        out_ref[0, pl.ds(i, 1), :] = prob_s[pl.ds(r, 1), :]
        return 0

    jax.lax.fori_loop(0, _KR, copy_body, 0)


_NW = 25
_ICH = 8


def _rowstat_kernel(cand_ref, rows_ref, m_ref, a_ref):
    p = cand_ref[...]
    rows = rows_ref[...]
    lane_t3 = jax.lax.broadcasted_iota(jnp.int32, (_ICH, _KR, _T), 2)
    m0 = jnp.max(p, axis=2)
    col = jnp.min(jnp.where(p == m0[:, :, None], lane_t3, _T), axis=2)
    m_ref[...] = m0
    a_ref[...] = rows * _T + col


def _final_kernel(candf_ref, m_ref, a_ref, rowsf_ref,
                  scores_ref, ix_ref, pl1_ref, pl2_ref, pl3_ref):
    n = _B
    pf0 = candf_ref[...]
    b1 = pf0.astype(jnp.bfloat16)
    r1 = pf0 - b1.astype(jnp.float32)
    b2 = r1.astype(jnp.bfloat16)
    b3 = (r1 - b2.astype(jnp.float32)).astype(jnp.bfloat16)
    pl1_ref[...] = b1
    pl2_ref[...] = b2
    pl3_ref[...] = b3

    m0 = m_ref[...]
    a0 = a_ref[...]
    rowsf = rowsf_ref[...]
    lane50 = jax.lax.broadcasted_iota(jnp.int32, (n, _K), 1)
    lane_t = jax.lax.broadcasted_iota(jnp.int32, (n, _T), 1)
    w_iota = jax.lax.broadcasted_iota(jnp.int32, (n, _KR, _NW), 2)
    big = _Q * _T

    g_iota = jax.lax.broadcasted_iota(jnp.int32, (n, n * _KR), 1)
    b_iota = jax.lax.broadcasted_iota(jnp.int32, (n, n * _KR), 0)
    eqb = (g_iota // _KR) == b_iota

    dn = (((1,), (0,)), ((), ()))

    def topk_body(i, carry):
        m, a, taken, sc, ix = carry
        mx = jnp.max(m, axis=1, keepdims=True)
        sel = jnp.min(jnp.where(m == mx, a, big), axis=1, keepdims=True)
        sc = jnp.where(lane50 == i, mx, sc)
        ix = jnp.where(lane50 == i, sel, ix)
        selrow = sel // _T
        selcol = sel - selrow * _T
        jhot = (m == mx) & (a == sel)
        jhotf = (eqb & (rowsf == selrow)).astype(jnp.bfloat16)
        xrow = (jax.lax.dot_general(jhotf, pl1_ref[...], dn,
                                    preferred_element_type=jnp.float32)
                + jax.lax.dot_general(jhotf, pl2_ref[...], dn,
                                      preferred_element_type=jnp.float32)
                + jax.lax.dot_general(jhotf, pl3_ref[...], dn,
                                      preferred_element_type=jnp.float32))
        jhot3 = (m[:, :, None] == mx[:, :, None]) & (a[:, :, None] == sel[:, :, None])
        tw = jnp.sum(jnp.where(jhot3, taken, 0), axis=1)
        exp = jnp.broadcast_to(tw[:, :, None], (n, _NW, 32))
        exp = exp.reshape(n, _NW * 32)[:, :_T]
        bit = jax.lax.shift_right_logical(exp, lane_t % 32) & 1
        dead = (bit == 1) | (lane_t == selcol)
        xm = jnp.where(dead, -jnp.inf, xrow)
        newmax = jnp.max(xm, axis=1, keepdims=True)
        flatx = selrow * _T + lane_t
        newa = jnp.min(jnp.where(xm == newmax, flatx, big), axis=1,
                       keepdims=True)
        m = jnp.where(jhot, newmax, m)
        a = jnp.where(jhot, newa, a)
        setmask = jhot3 & (w_iota == (selcol // 32)[:, :, None])
        bitval = jax.lax.shift_left(jnp.int32(1), (selcol % 32)[:, :, None])
        taken = taken | jnp.where(setmask, bitval, 0)
        return m, a, taken, sc, ix

    _, _, _, sc, ix = jax.lax.fori_loop(
        0, _K, topk_body,
        (m0, a0, jnp.zeros((n, _KR, _NW), jnp.int32),
         jnp.zeros((n, _K), jnp.float32), jnp.zeros((n, _K), jnp.int32)))

    scores_ref[...] = sc
    ix_ref[...] = ix


def _gather_kernel(ix_ref, boxes_ref, ts_ref, cls_ref,
                   labels_ref, boxes_out_ref):
    n = _ICH
    ix = ix_ref[...]
    j = ix // _T
    lab = ix - j * _T

    table = cls_ref[...]
    pb = boxes_ref[...]
    cx, cy, w, h = pb[:, 0, :], pb[:, 1, :], pb[:, 2, :], pb[:, 3, :]
    x0 = cx - 0.5 * w
    y0 = cy - 0.5 * h
    x1 = cx + 0.5 * w
    y1 = cy + 0.5 * h
    ts = ts_ref[...].astype(jnp.float32)
    ih = ts[:, 0:1]
    iw = ts[:, 1:2]

    q_iota = jax.lax.broadcasted_iota(jnp.int32, (n, _K, _Q), 2)
    t_iota = jax.lax.broadcasted_iota(jnp.int32, (n, _K, _T), 2)
    mq = q_iota == j[:, :, None]
    mt = t_iota == lab[:, :, None]
    cls_a = jnp.sum(jnp.where(mt, table[:, None, :], 0), axis=2)
    b0 = jnp.sum(jnp.where(mq, x0[:, None, :], 0.0), axis=2) * iw
    b1 = jnp.sum(jnp.where(mq, y0[:, None, :], 0.0), axis=2) * ih
    b2 = jnp.sum(jnp.where(mq, x1[:, None, :], 0.0), axis=2) * iw
    b3 = jnp.sum(jnp.where(mq, y1[:, None, :], 0.0), axis=2) * ih

    labels_ref[...] = cls_a
    boxes_out_ref[...] = jnp.stack([b0, b1, b2, b3], axis=-1)


def kernel(pred_logits, pred_boxes, target_sizes, positive_map):
    rmax = pl.pallas_call(
        _rowmax_kernel,
        grid=(_B,),
        in_specs=[
            pl.BlockSpec((1, _Q, _C), lambda b: (b, 0, 0)),
            pl.BlockSpec((_T, _C), lambda b: (0, 0)),
        ],
        out_specs=pl.BlockSpec((1, _Q, 1), lambda b: (b, 0, 0)),
        out_shape=jax.ShapeDtypeStruct((_B, _Q, 1), jnp.float32),
    )(pred_logits, positive_map)

    rows = pl.pallas_call(
        _row_topk_kernel,
        in_specs=[pl.BlockSpec((_B, _Q), lambda: (0, 0))],
        out_specs=pl.BlockSpec((_B, _KR), lambda: (0, 0)),
        out_shape=jax.ShapeDtypeStruct((_B, _KR), jnp.int32),
    )(rmax.reshape(_B, _Q))

    cand = pl.pallas_call(
        _cand_prob_kernel,
        grid_spec=pltpu.PrefetchScalarGridSpec(
            num_scalar_prefetch=1,
            grid=(_B,),
            in_specs=[
                pl.BlockSpec((1, _Q, _C), lambda b, sref: (b, 0, 0)),
                pl.BlockSpec((_T, _C), lambda b, sref: (0, 0)),
            ],
            out_specs=pl.BlockSpec((1, _KR, _T), lambda b, sref: (b, 0, 0)),
            scratch_shapes=[pltpu.VMEM((_Q, _T), jnp.float32)],
        ),
        out_shape=jax.ShapeDtypeStruct((_B, _KR, _T), jnp.float32),
    )(rows.reshape(_B * _KR), pred_logits, positive_map)

    m0, a0 = pl.pallas_call(
        _rowstat_kernel,
        grid=(_B // _ICH,),
        in_specs=[
            pl.BlockSpec((_ICH, _KR, _T), lambda b: (b, 0, 0)),
            pl.BlockSpec((_ICH, _KR), lambda b: (b, 0)),
        ],
        out_specs=[
            pl.BlockSpec((_ICH, _KR), lambda b: (b, 0)),
            pl.BlockSpec((_ICH, _KR), lambda b: (b, 0)),
        ],
        out_shape=[
            jax.ShapeDtypeStruct((_B, _KR), jnp.float32),
            jax.ShapeDtypeStruct((_B, _KR), jnp.int32),
        ],
    )(cand, rows)

    scores, ix = pl.pallas_call(
        _final_kernel,
        in_specs=[
            pl.BlockSpec((_B * _KR, _T), lambda: (0, 0)),
            pl.BlockSpec((_B, _KR), lambda: (0, 0)),
            pl.BlockSpec((_B, _KR), lambda: (0, 0)),
            pl.BlockSpec((1, _B * _KR), lambda: (0, 0)),
        ],
        out_specs=[
            pl.BlockSpec((_B, _K), lambda: (0, 0)),
            pl.BlockSpec((_B, _K), lambda: (0, 0)),
        ],
        out_shape=[
            jax.ShapeDtypeStruct((_B, _K), jnp.float32),
            jax.ShapeDtypeStruct((_B, _K), jnp.int32),
        ],
        scratch_shapes=[
            pltpu.VMEM((_B * _KR, _T), jnp.bfloat16),
            pltpu.VMEM((_B * _KR, _T), jnp.bfloat16),
            pltpu.VMEM((_B * _KR, _T), jnp.bfloat16),
        ],
    )(cand.reshape(_B * _KR, _T), m0, a0, rows.reshape(1, _B * _KR))

    cls_table = jnp.asarray(_CLS_TABLE).reshape(1, _T)
    boxes_t = jnp.transpose(pred_boxes, (0, 2, 1))
    labels, boxes = pl.pallas_call(
        _gather_kernel,
        grid=(_B // _ICH,),
        in_specs=[
            pl.BlockSpec((_ICH, _K), lambda b: (b, 0)),
            pl.BlockSpec((_ICH, 4, _Q), lambda b: (b, 0, 0)),
            pl.BlockSpec((_ICH, 2), lambda b: (b, 0)),
            pl.BlockSpec((1, _T), lambda b: (0, 0)),
        ],
        out_specs=[
            pl.BlockSpec((_ICH, _K), lambda b: (b, 0)),
            pl.BlockSpec((_ICH, _K, 4), lambda b: (b, 0, 0)),
        ],
        out_shape=[
            jax.ShapeDtypeStruct((_B, _K), jnp.int32),
            jax.ShapeDtypeStruct((_B, _K, 4), jnp.float32),
        ],
    )(ix, boxes_t, target_sizes, cls_table)

    return scores, labels, boxes


# speedup vs baseline: 11.5529x; 1.1508x over previous
import numpy as np
import jax
import jax.numpy as jnp
from jax.experimental import pallas as pl
from jax.experimental.pallas import tpu as pltpu

_TOKEN_IDX = np.array([0, 9, 19, 25, 38, 49, 55, 63, 71, 78, 94, 109, 121,
                       137, 145, 152, 158, 164, 172, 180, 186, 197, 204, 212,
                       222, 233, 244, 254, 260, 271, 281, 288, 300, 314, 321,
                       336, 353, 366, 378, 394, 403, 416, 422, 429, 437, 445,
                       452, 461, 469, 480, 489, 500, 509, 519, 527, 535, 542,
                       550, 558, 573, 579, 594, 603, 608, 617, 625, 634, 645,
                       658, 670, 677, 687, 694, 709, 716, 724, 731, 742, 755,
                       768], dtype=np.int64)
_MAX_TOKEN = 768
_LOOKUP = np.full(_MAX_TOKEN + 1, -1, dtype=np.int64)
_LOOKUP[_TOKEN_IDX] = np.arange(len(_TOKEN_IDX), dtype=np.int64)
_CLS_TABLE = np.maximum(_LOOKUP, 0).astype(np.int32)

_B = 64
_Q = 900
_C = 256
_T = _MAX_TOKEN + 1
_KR = 64
_K = 50
_DCH = 8


_ACH = 4


def _rowmax_kernel(logits_ref, pm_ref, rmax_ref):
    for c in range(_ACH):
        x = jax.nn.sigmoid(logits_ref[c])
        prob = jax.lax.dot_general(x, pm_ref[...], (((1,), (1,)), ((), ())),
                                   preferred_element_type=jnp.float32)
        rmax_ref[c] = jnp.max(prob, axis=1, keepdims=True)


def _row_topk_kernel(rmax_ref, rows_ref):
    m0 = rmax_ref[...]
    iota = jax.lax.broadcasted_iota(jnp.int32, (_B, _Q), 1)
    lane = jax.lax.broadcasted_iota(jnp.int32, (_B, _KR), 1)

    def body(i, carry):
        m, rows = carry
        mx = jnp.max(m, axis=1, keepdims=True)
        idx = jnp.min(jnp.where(m == mx, iota, _Q), axis=1, keepdims=True)
        rows = jnp.where(lane == i, idx, rows)
        m = jnp.where(iota == idx, -jnp.inf, m)
        return m, rows

    _, rows = jax.lax.fori_loop(
        0, _KR, body, (m0, jnp.zeros((_B, _KR), jnp.int32)))
    rows_ref[...] = rows


def _cand_prob_kernel(rows_sref, logits_ref, pm_ref, out_ref, prob_s):
    step = pl.program_id(0)
    for c in range(_ACH):
        x = jax.nn.sigmoid(logits_ref[c])
        prob_s[c] = jax.lax.dot_general(x, pm_ref[...], (((1,), (1,)), ((), ())),
                                        preferred_element_type=jnp.float32)

    def copy_body(i, _):
        for c in range(_ACH):
            r = rows_sref[(step * _ACH + c) * _KR + i]
            out_ref[c, pl.ds(i, 1), :] = prob_s[c, pl.ds(r, 1), :]
        return 0

    jax.lax.fori_loop(0, _KR, copy_body, 0)


_NW = 25
_ICH = 8


def _rowstat_kernel(cand_ref, rows_ref, m_ref, a_ref):
    p = cand_ref[...]
    rows = rows_ref[...]
    lane_t3 = jax.lax.broadcasted_iota(jnp.int32, (_ICH, _KR, _T), 2)
    m0 = jnp.max(p, axis=2)
    col = jnp.min(jnp.where(p == m0[:, :, None], lane_t3, _T), axis=2)
    m_ref[...] = m0
    a_ref[...] = rows * _T + col


def _final_kernel(candf_ref, m_ref, a_ref, rowsf_ref,
                  scores_ref, ix_ref, pl1_ref, pl2_ref, pl3_ref):
    n = _B
    pf0 = candf_ref[...]
    b1 = pf0.astype(jnp.bfloat16)
    r1 = pf0 - b1.astype(jnp.float32)
    b2 = r1.astype(jnp.bfloat16)
    b3 = (r1 - b2.astype(jnp.float32)).astype(jnp.bfloat16)
    pl1_ref[...] = b1
    pl2_ref[...] = b2
    pl3_ref[...] = b3

    m0 = m_ref[...]
    a0 = a_ref[...]
    rowsf = rowsf_ref[...]
    lane50 = jax.lax.broadcasted_iota(jnp.int32, (n, _K), 1)
    lane_t = jax.lax.broadcasted_iota(jnp.int32, (n, _T), 1)
    w_iota = jax.lax.broadcasted_iota(jnp.int32, (n, _KR, _NW), 2)
    big = _Q * _T

    g_iota = jax.lax.broadcasted_iota(jnp.int32, (n, n * _KR), 1)
    b_iota = jax.lax.broadcasted_iota(jnp.int32, (n, n * _KR), 0)
    eqb = (g_iota // _KR) == b_iota

    dn = (((1,), (0,)), ((), ()))

    def topk_body(i, carry):
        m, a, taken, sc, ix = carry
        mx = jnp.max(m, axis=1, keepdims=True)
        sel = jnp.min(jnp.where(m == mx, a, big), axis=1, keepdims=True)
        sc = jnp.where(lane50 == i, mx, sc)
        ix = jnp.where(lane50 == i, sel, ix)
        selrow = sel // _T
        selcol = sel - selrow * _T
        jhot = (m == mx) & (a == sel)
        jhotf = (eqb & (rowsf == selrow)).astype(jnp.bfloat16)
        xrow = (jax.lax.dot_general(jhotf, pl1_ref[...], dn,
                                    preferred_element_type=jnp.float32)
                + jax.lax.dot_general(jhotf, pl2_ref[...], dn,
                                      preferred_element_type=jnp.float32)
                + jax.lax.dot_general(jhotf, pl3_ref[...], dn,
                                      preferred_element_type=jnp.float32))
        jhot3 = (m[:, :, None] == mx[:, :, None]) & (a[:, :, None] == sel[:, :, None])
        tw = jnp.sum(jnp.where(jhot3, taken, 0), axis=1)
        exp = jnp.broadcast_to(tw[:, :, None], (n, _NW, 32))
        exp = exp.reshape(n, _NW * 32)[:, :_T]
        bit = jax.lax.shift_right_logical(exp, lane_t % 32) & 1
        dead = (bit == 1) | (lane_t == selcol)
        xm = jnp.where(dead, -jnp.inf, xrow)
        newmax = jnp.max(xm, axis=1, keepdims=True)
        flatx = selrow * _T + lane_t
        newa = jnp.min(jnp.where(xm == newmax, flatx, big), axis=1,
                       keepdims=True)
        m = jnp.where(jhot, newmax, m)
        a = jnp.where(jhot, newa, a)
        setmask = jhot3 & (w_iota == (selcol // 32)[:, :, None])
        bitval = jax.lax.shift_left(jnp.int32(1), (selcol % 32)[:, :, None])
        taken = taken | jnp.where(setmask, bitval, 0)
        return m, a, taken, sc, ix

    _, _, _, sc, ix = jax.lax.fori_loop(
        0, _K, topk_body,
        (m0, a0, jnp.zeros((n, _KR, _NW), jnp.int32),
         jnp.zeros((n, _K), jnp.float32), jnp.zeros((n, _K), jnp.int32)))

    scores_ref[...] = sc
    ix_ref[...] = ix


def _gather_kernel(ix_ref, boxes_ref, ts_ref, cls_ref,
                   labels_ref, boxes_out_ref):
    n = _ICH
    ix = ix_ref[...]
    j = ix // _T
    lab = ix - j * _T

    table = cls_ref[...]
    pb = boxes_ref[...]
    cx, cy, w, h = pb[:, 0, :], pb[:, 1, :], pb[:, 2, :], pb[:, 3, :]
    x0 = cx - 0.5 * w
    y0 = cy - 0.5 * h
    x1 = cx + 0.5 * w
    y1 = cy + 0.5 * h
    ts = ts_ref[...].astype(jnp.float32)
    ih = ts[:, 0:1]
    iw = ts[:, 1:2]

    q_iota = jax.lax.broadcasted_iota(jnp.int32, (n, _K, _Q), 2)
    t_iota = jax.lax.broadcasted_iota(jnp.int32, (n, _K, _T), 2)
    mq = q_iota == j[:, :, None]
    mt = t_iota == lab[:, :, None]
    cls_a = jnp.sum(jnp.where(mt, table[:, None, :], 0), axis=2)
    b0 = jnp.sum(jnp.where(mq, x0[:, None, :], 0.0), axis=2) * iw
    b1 = jnp.sum(jnp.where(mq, y0[:, None, :], 0.0), axis=2) * ih
    b2 = jnp.sum(jnp.where(mq, x1[:, None, :], 0.0), axis=2) * iw
    b3 = jnp.sum(jnp.where(mq, y1[:, None, :], 0.0), axis=2) * ih

    labels_ref[...] = cls_a
    boxes_out_ref[...] = jnp.stack([b0, b1, b2, b3], axis=-1)


def kernel(pred_logits, pred_boxes, target_sizes, positive_map):
    rmax = pl.pallas_call(
        _rowmax_kernel,
        grid=(_B // _ACH,),
        in_specs=[
            pl.BlockSpec((_ACH, _Q, _C), lambda b: (b, 0, 0)),
            pl.BlockSpec((_T, _C), lambda b: (0, 0)),
        ],
        out_specs=pl.BlockSpec((_ACH, _Q, 1), lambda b: (b, 0, 0)),
        out_shape=jax.ShapeDtypeStruct((_B, _Q, 1), jnp.float32),
    )(pred_logits, positive_map)

    rows = pl.pallas_call(
        _row_topk_kernel,
        in_specs=[pl.BlockSpec((_B, _Q), lambda: (0, 0))],
        out_specs=pl.BlockSpec((_B, _KR), lambda: (0, 0)),
        out_shape=jax.ShapeDtypeStruct((_B, _KR), jnp.int32),
    )(rmax.reshape(_B, _Q))

    cand = pl.pallas_call(
        _cand_prob_kernel,
        grid_spec=pltpu.PrefetchScalarGridSpec(
            num_scalar_prefetch=1,
            grid=(_B // _ACH,),
            in_specs=[
                pl.BlockSpec((_ACH, _Q, _C), lambda b, sref: (b, 0, 0)),
                pl.BlockSpec((_T, _C), lambda b, sref: (0, 0)),
            ],
            out_specs=pl.BlockSpec((_ACH, _KR, _T), lambda b, sref: (b, 0, 0)),
            scratch_shapes=[pltpu.VMEM((_ACH, _Q, _T), jnp.float32)],
        ),
        out_shape=jax.ShapeDtypeStruct((_B, _KR, _T), jnp.float32),
    )(rows.reshape(_B * _KR), pred_logits, positive_map)

    m0, a0 = pl.pallas_call(
        _rowstat_kernel,
        grid=(_B // _ICH,),
        in_specs=[
            pl.BlockSpec((_ICH, _KR, _T), lambda b: (b, 0, 0)),
            pl.BlockSpec((_ICH, _KR), lambda b: (b, 0)),
        ],
        out_specs=[
            pl.BlockSpec((_ICH, _KR), lambda b: (b, 0)),
            pl.BlockSpec((_ICH, _KR), lambda b: (b, 0)),
        ],
        out_shape=[
            jax.ShapeDtypeStruct((_B, _KR), jnp.float32),
            jax.ShapeDtypeStruct((_B, _KR), jnp.int32),
        ],
    )(cand, rows)

    scores, ix = pl.pallas_call(
        _final_kernel,
        in_specs=[
            pl.BlockSpec((_B * _KR, _T), lambda: (0, 0)),
            pl.BlockSpec((_B, _KR), lambda: (0, 0)),
            pl.BlockSpec((_B, _KR), lambda: (0, 0)),
            pl.BlockSpec((1, _B * _KR), lambda: (0, 0)),
        ],
        out_specs=[
            pl.BlockSpec((_B, _K), lambda: (0, 0)),
            pl.BlockSpec((_B, _K), lambda: (0, 0)),
        ],
        out_shape=[
            jax.ShapeDtypeStruct((_B, _K), jnp.float32),
            jax.ShapeDtypeStruct((_B, _K), jnp.int32),
        ],
        scratch_shapes=[
            pltpu.VMEM((_B * _KR, _T), jnp.bfloat16),
            pltpu.VMEM((_B * _KR, _T), jnp.bfloat16),
            pltpu.VMEM((_B * _KR, _T), jnp.bfloat16),
        ],
    )(cand.reshape(_B * _KR, _T), m0, a0, rows.reshape(1, _B * _KR))

    cls_table = jnp.asarray(_CLS_TABLE).reshape(1, _T)
    boxes_t = jnp.transpose(pred_boxes, (0, 2, 1))
    labels, boxes = pl.pallas_call(
        _gather_kernel,
        grid=(_B // _ICH,),
        in_specs=[
            pl.BlockSpec((_ICH, _K), lambda b: (b, 0)),
            pl.BlockSpec((_ICH, 4, _Q), lambda b: (b, 0, 0)),
            pl.BlockSpec((_ICH, 2), lambda b: (b, 0)),
            pl.BlockSpec((1, _T), lambda b: (0, 0)),
        ],
        out_specs=[
            pl.BlockSpec((_ICH, _K), lambda b: (b, 0)),
            pl.BlockSpec((_ICH, _K, 4), lambda b: (b, 0, 0)),
        ],
        out_shape=[
            jax.ShapeDtypeStruct((_B, _K), jnp.int32),
            jax.ShapeDtypeStruct((_B, _K, 4), jnp.float32),
        ],
    )(ix, boxes_t, target_sizes, cls_table)

    return scores, labels, boxes
